# baseline (speedup 1.0000x reference)
import functools

import jax
import jax.numpy as jnp
from jax import lax
from jax.experimental import pallas as pl
from jax.experimental.pallas import tpu as pltpu, tpu_sc as plsc

_NC, _NS, _LANES = 2, 16, 16
_NW = _NC * _NS
_B, _L, _D = 4096, 200, 128
_LD = _L * _D
_ROWS = _B // _NW
_CH = 2
_NCHUNK = _ROWS // _CH
_CD = _CH * _LD
_NBUF = 2
_NV = _CD // _LANES


def _sc_body(x_hbm, pos_hbm, out_hbm, pos_v, buf0, buf1, in_sems, out_sems):
    wid = lax.axis_index("s") * _NC + lax.axis_index("c")
    base = wid * _ROWS * _LD
    bufs = (buf0, buf1)

    pltpu.sync_copy(pos_hbm, pos_v)

    def start_in(g, b):
        pltpu.make_async_copy(
            x_hbm.at[pl.ds(base + g * _CD, _CD)], bufs[b], in_sems.at[b]).start()

    def wait_in(b):
        pltpu.make_async_copy(
            x_hbm.at[pl.ds(base, _CD)], bufs[b], in_sems.at[b]).wait()

    def start_out(g, b):
        pltpu.make_async_copy(
            bufs[b], out_hbm.at[pl.ds(base + g * _CD, _CD)], out_sems.at[b]).start()

    def wait_out(b):
        pltpu.make_async_copy(
            bufs[b], out_hbm.at[pl.ds(base, _CD)], out_sems.at[b]).wait()

    def compute(b):
        for r in range(_CH):
            @pl.loop(0, _LD // _LANES, unroll=8)
            def _(i, r=r):
                off = i * _LANES
                plsc.addupdate(bufs[b].at[pl.ds(r * _LD + off, _LANES)],
                               pos_v[pl.ds(off, _LANES)])

    for g in range(_NBUF - 1):
        start_in(g, g)

    for g in range(_NBUF):
        wait_in(g)
        compute(g)
        start_out(g, g)
        bn = (g + _NBUF - 1) % _NBUF
        if g >= 1:
            wait_out(bn)
        start_in(g + _NBUF - 1, bn)

    _K = (_NCHUNK + 1 - 2 * _NBUF) // _NBUF

    @pl.loop(0, _K)
    def _(i):
        for b in range(_NBUF):
            g = _NBUF + i * _NBUF + b
            wait_in(b)
            compute(b)
            start_out(g, b)
            bn = (b + _NBUF - 1) % _NBUF
            wait_out(bn)
            start_in(g + _NBUF - 1, bn)

    for g in range(_NBUF + _K * _NBUF, _NCHUNK):
        b = g % _NBUF
        wait_in(b)
        compute(b)
        start_out(g, b)
        bn = (b + _NBUF - 1) % _NBUF
        wait_out(bn)
        if g + _NBUF - 1 < _NCHUNK:
            start_in(g + _NBUF - 1, bn)

    wait_out((_NCHUNK - 1) % _NBUF)


@jax.jit
def _sc_add(x1, pos1):
    body = functools.partial(
        pl.kernel,
        out_type=jax.ShapeDtypeStruct((_B * _LD,), jnp.float32),
        mesh=plsc.VectorSubcoreMesh(
            core_axis_name="c", subcore_axis_name="s",
            num_cores=_NC, num_subcores=_NS,
        ),
        scratch_types=[
            pltpu.VMEM((_LD,), jnp.float32),
            pltpu.VMEM((_CD,), jnp.float32),
            pltpu.VMEM((_CD,), jnp.float32),
            pltpu.SemaphoreType.DMA((_NBUF,)),
            pltpu.SemaphoreType.DMA((_NBUF,)),
        ],
    )(_sc_body)
    return body(x1, pos1)


def kernel(x, pos_embedding):
    Bx, Lx, Dx = x.shape
    out1 = _sc_add(x.reshape(Bx * Lx * Dx), pos_embedding.reshape(Lx * Dx))
    return out1.reshape(Bx, Lx, Dx)


# speedup vs baseline: 1.4666x; 1.4666x over previous
import functools

import jax
import jax.numpy as jnp
from jax import lax
from jax.experimental import pallas as pl
from jax.experimental.pallas import tpu as pltpu, tpu_sc as plsc

_NC, _NS, _LANES = 2, 16, 16
_NW = _NC * _NS
_B, _L, _D = 4096, 200, 128
_LD = _L * _D
_ROWS = _B // _NW
_NBUF = 3
_NV = _LD // _LANES


def _sc_body(x_hbm, pos_hbm, out_hbm, pos_v, buf0, buf1, buf2,
             in_sems, out_sems):
    wid = lax.axis_index("s") * _NC + lax.axis_index("c")
    base = wid * _ROWS
    bufs = (buf0, buf1, buf2)

    def start_in(g, b):
        pltpu.make_async_copy(
            x_hbm.at[pl.ds((base + g) * _LD, _LD)], bufs[b], in_sems.at[b]).start()

    def wait_in(b):
        pltpu.make_async_copy(
            x_hbm.at[pl.ds(base * _LD, _LD)], bufs[b], in_sems.at[b]).wait()

    def start_out(g, b):
        pltpu.make_async_copy(
            bufs[b], out_hbm.at[pl.ds((base + g) * _LD, _LD)], out_sems.at[b]).start()

    def wait_out(b):
        pltpu.make_async_copy(
            bufs[b], out_hbm.at[pl.ds(base * _LD, _LD)], out_sems.at[b]).wait()

    def compute(b):
        @pl.loop(0, _NV, unroll=8)
        def _(i):
            off = i * _LANES
            plsc.addupdate(bufs[b].at[pl.ds(off, _LANES)], pos_v[pl.ds(off, _LANES)])

    for g in range(_NBUF - 1):
        start_in(g, g)
    pltpu.sync_copy(pos_hbm, pos_v)

    for g in range(_NBUF):
        wait_in(g)
        compute(g)
        start_out(g, g)
        bn = (g + _NBUF - 1) % _NBUF
        if g >= 1:
            wait_out(bn)
        start_in(g + _NBUF - 1, bn)

    _K = (_ROWS + 1 - 2 * _NBUF) // _NBUF

    @pl.loop(0, _K)
    def _(i):
        for b in range(_NBUF):
            g = _NBUF + i * _NBUF + b
            wait_in(b)
            compute(b)
            start_out(g, b)
            bn = (b + _NBUF - 1) % _NBUF
            wait_out(bn)
            start_in(g + _NBUF - 1, bn)

    for g in range(_NBUF + _K * _NBUF, _ROWS):
        b = g % _NBUF
        wait_in(b)
        compute(b)
        start_out(g, b)
        bn = (b + _NBUF - 1) % _NBUF
        wait_out(bn)
        if g + _NBUF - 1 < _ROWS:
            start_in(g + _NBUF - 1, bn)

    wait_out((_ROWS - 1) % _NBUF)


@jax.jit
def _sc_add(x1, pos1):
    body = functools.partial(
        pl.kernel,
        out_type=jax.ShapeDtypeStruct((_B * _LD,), jnp.float32),
        mesh=plsc.VectorSubcoreMesh(
            core_axis_name="c", subcore_axis_name="s",
            num_cores=_NC, num_subcores=_NS,
        ),
        scratch_types=[
            pltpu.VMEM((_LD,), jnp.float32),
            pltpu.VMEM((_LD,), jnp.float32),
            pltpu.VMEM((_LD,), jnp.float32),
            pltpu.VMEM((_LD,), jnp.float32),
            pltpu.SemaphoreType.DMA((_NBUF,)),
            pltpu.SemaphoreType.DMA((_NBUF,)),
        ],
    )(_sc_body)
    return body(x1, pos1)


def kernel(x, pos_embedding):
    Bx, Lx, Dx = x.shape
    out1 = _sc_add(x.reshape(Bx * Lx * Dx), pos_embedding.reshape(Lx * Dx))
    return out1.reshape(Bx, Lx, Dx)
